# Initial kernel scaffold; baseline (speedup 1.0000x reference)
import jax
import jax.numpy as jnp
from jax.experimental import pallas as pl


def kernel(inputs_ids, labels, labels_mask, valid_mask, params):
    raise NotImplementedError("write your pallas kernel here")


# speedup vs baseline: 1.8202x; 1.8202x over previous
import functools

import jax
import jax.numpy as jnp
from jax.experimental import pallas as pl

F32 = jnp.float32


def _ln(x, g, b):
    m = x.mean(-1, keepdims=True)
    v = ((x - m) ** 2).mean(-1, keepdims=True)
    return (x - m) / jnp.sqrt(v + 1e-6) * g + b


def _fwd_kernel(
    ids_m_ref, ids_h_ref, pos_m_ref, pos_h_ref, emb_ref,
    wq_ref, bq_ref, wk_ref, bk_ref, wv_ref, bv_ref, wo_ref, bo_ref,
    g1_ref, b1_ref, g2_ref, b2_ref,
    w1_ref, bf1_ref, w2_ref, bf2_ref,
    lmw_ref, lmb_ref, l1w_ref, l1b_ref, chw_ref, chb_ref,
    lbl_oh_ref, w_ref,
    out_ref,
    *, T, CH, H, DH, V, NL,
):
    s_blk = pl.program_id(0)
    TE = T + CH

    ids_ext = jnp.concatenate([ids_h_ref[0], ids_m_ref[0]], axis=0)
    pos_ext = jnp.concatenate([pos_h_ref[0], pos_m_ref[0]], axis=0)
    onehot = (ids_ext == jax.lax.broadcasted_iota(jnp.int32, (TE, V), 1)
              ).astype(F32)
    x_ext = jnp.dot(onehot, emb_ref[...], preferred_element_type=F32) + pos_ext

    h1 = _ln(x_ext, g1_ref[...], b1_ref[...])
    q_all = jnp.dot(h1[CH:], wq_ref[...], preferred_element_type=F32) + bq_ref[...]
    k_all = jnp.dot(h1, wk_ref[...], preferred_element_type=F32) + bk_ref[...]
    v_all = jnp.dot(h1, wv_ref[...], preferred_element_type=F32) + bv_ref[...]

    ti = jax.lax.broadcasted_iota(jnp.int32, (T, TE), 0)
    ji = jax.lax.broadcasted_iota(jnp.int32, (T, TE), 1)
    tc = ti // CH
    jc = ji // CH
    band = (jc == tc) | (jc == tc + 1)
    causal = ji <= ti + CH
    first = (s_blk * (T // CH) + tc) > 0
    mask = band & causal & (first | (ji >= CH))

    scale = 1.0 / (DH ** 0.5)
    outs = []
    for h in range(H):
        sl = slice(h * DH, (h + 1) * DH)
        qh = q_all[:, sl]
        kh = k_all[:, sl]
        vh = v_all[:, sl]
        sc = jax.lax.dot_general(qh, kh, (((1,), (1,)), ((), ())),
                                 preferred_element_type=F32) * scale
        sc = jnp.where(mask, sc, -1e9)
        mx = jnp.max(sc, axis=-1, keepdims=True)
        p = jnp.exp(sc - mx)
        p = p / jnp.sum(p, axis=-1, keepdims=True)
        outs.append(jnp.dot(p, vh, preferred_element_type=F32))
    o = jnp.concatenate(outs, axis=-1)

    x = x_ext[CH:] + jnp.dot(o, wo_ref[...], preferred_element_type=F32) + bo_ref[...]

    h2 = _ln(x, g2_ref[...], b2_ref[...])
    ff = jnp.maximum(jnp.dot(h2, w1_ref[...], preferred_element_type=F32)
                     + bf1_ref[...], 0.0)
    x = x + jnp.dot(ff, w2_ref[...], preferred_element_type=F32) + bf2_ref[...]

    lm = jnp.dot(x, lmw_ref[...], preferred_element_type=F32) + lmb_ref[...]
    o1 = jnp.maximum(jnp.dot(lm, l1w_ref[...], preferred_element_type=F32)
                     + l1b_ref[...], 0.0)
    logits = jnp.dot(o1, chw_ref[...], preferred_element_type=F32) + chb_ref[...]

    mx = jnp.max(logits, axis=-1, keepdims=True)
    lse = mx + jnp.log(jnp.sum(jnp.exp(logits - mx), axis=-1, keepdims=True))
    sel = jnp.sum(lbl_oh_ref[0] * logits, axis=-1, keepdims=True)
    wcol = w_ref[0]
    nll_sum = jnp.sum((lse - sel) * wcol)
    w_sum = jnp.sum(wcol)

    lane = jax.lax.broadcasted_iota(jnp.int32, (1, 1, 128), 2)
    out_ref[...] = jnp.where(lane == 0, nll_sum,
                             jnp.where(lane == 1, w_sum, 0.0)).astype(F32)


def kernel(inputs_ids, labels, labels_mask, valid_mask, params):
    B, S = inputs_ids.shape
    D = params['Wq'].shape[0]
    V, NL = params['chW'].shape
    FF = params['W1'].shape[1]
    H = 8
    DH = D // H
    CH = 64
    T = 512
    NS = S // T
    A1 = params['pos1'].shape[0]
    A2 = params['pos2'].shape[1]

    pos = jnp.concatenate([
        jnp.broadcast_to(params['pos1'], (A1, A2, D // 2)),
        jnp.broadcast_to(params['pos2'], (A1, A2, D // 2)),
    ], axis=-1).reshape(S, D)

    ids = inputs_ids.astype(jnp.int32)
    ids_m = ids.reshape(B * NS, T, 1)
    ids_h = jnp.pad(ids, ((0, 0), (CH, 0)))[:, :S].reshape(B, NS, T)[:, :, :CH]
    ids_h = ids_h.reshape(B * NS, CH, 1)
    pos_m = pos.reshape(NS, T, D)
    pos_h = jnp.pad(pos, ((CH, 0), (0, 0)))[:S].reshape(NS, T, D)[:, :CH]

    lbl_oh = (labels[..., None] == jnp.arange(NL)).astype(F32)
    lbl_oh = lbl_oh.reshape(B * NS, T, NL)
    w = ((valid_mask == 1) & (labels != 0)).astype(F32).reshape(B * NS, T, 1)

    r2 = lambda a: a.reshape(1, -1)
    p = params

    grid = (NS, B)
    in_specs = [
        pl.BlockSpec((1, T, 1), lambda s, b: (b * NS + s, 0, 0)),
        pl.BlockSpec((1, CH, 1), lambda s, b: (b * NS + s, 0, 0)),
        pl.BlockSpec((1, T, D), lambda s, b: (s, 0, 0)),
        pl.BlockSpec((1, CH, D), lambda s, b: (s, 0, 0)),
        pl.BlockSpec((V, D), lambda s, b: (0, 0)),
    ]
    const2d = lambda shape: pl.BlockSpec(shape, lambda s, b: (0, 0))
    weights = [
        (p['Wq'], (D, D)), (r2(p['bq']), (1, D)),
        (p['Wk'], (D, D)), (r2(p['bk']), (1, D)),
        (p['Wv'], (D, D)), (r2(p['bv']), (1, D)),
        (p['Wo'], (D, D)), (r2(p['bo']), (1, D)),
        (r2(p['g1']), (1, D)), (r2(p['b1']), (1, D)),
        (r2(p['g2']), (1, D)), (r2(p['b2']), (1, D)),
        (p['W1'], (D, FF)), (r2(p['bf1']), (1, FF)),
        (p['W2'], (FF, D)), (r2(p['bf2']), (1, D)),
        (p['lmW'], (D, V)), (r2(p['lmb']), (1, V)),
        (p['l1W'], (V, V)), (r2(p['l1b']), (1, V)),
        (p['chW'], (V, NL)), (r2(p['chb']), (1, NL)),
    ]
    in_specs += [const2d(shape) for _, shape in weights]
    in_specs += [
        pl.BlockSpec((1, T, NL), lambda s, b: (b * NS + s, 0, 0)),
        pl.BlockSpec((1, T, 1), lambda s, b: (b * NS + s, 0, 0)),
    ]

    out_spec = pl.BlockSpec((1, 1, 128), lambda s, b: (b * NS + s, 0, 0))

    fn = functools.partial(_fwd_kernel, T=T, CH=CH, H=H, DH=DH, V=V, NL=NL)
    partials = pl.pallas_call(
        fn,
        grid=grid,
        in_specs=in_specs,
        out_specs=out_spec,
        out_shape=jax.ShapeDtypeStruct((B * NS, 1, 128), F32),
    )(ids_m, ids_h, pos_m, pos_h, p['emb'],
      *[a for a, _ in weights], lbl_oh, w)

    nll_tot = jnp.sum(partials[:, 0, 0])
    w_tot = jnp.sum(partials[:, 0, 1])
    return nll_tot / jnp.maximum(w_tot, 1.0)
